# baseline (speedup 1.0000x reference)
import functools

import jax
import jax.numpy as jnp
from jax import lax
from jax.experimental import pallas as pl
from jax.experimental.pallas import tpu as pltpu
from jax.experimental.pallas import tpu_sc as plsc

_SC_NC = 2
_SC_NS = 16
_SC_NW = _SC_NC * _SC_NS
_CT = 32


def _sc_gather(table, idx):
    n = idx.shape[0]
    d = table.shape[1]
    b_per_w = n // _SC_NW
    chunk = min(128, b_per_w)
    n_chunks = b_per_w // chunk
    mesh = plsc.VectorSubcoreMesh(core_axis_name="c", subcore_axis_name="s")

    @functools.partial(
        pl.kernel, mesh=mesh,
        out_type=jax.ShapeDtypeStruct((n, d), jnp.float32),
        scratch_types=[
            pltpu.VMEM((chunk,), jnp.int32),
            pltpu.VMEM((chunk, d), jnp.float32),
            pltpu.SemaphoreType.DMA,
        ],
    )
    def k(table_hbm, idx_hbm, out_hbm, idx_v, rows_v, sem):
        wid = lax.axis_index("s") * _SC_NC + lax.axis_index("c")
        base = wid * b_per_w
        for j in range(n_chunks):
            off = base + j * chunk
            pltpu.sync_copy(idx_hbm.at[pl.ds(off, chunk)], idx_v)
            pltpu.async_copy(table_hbm.at[idx_v], rows_v, sem).wait()
            pltpu.sync_copy(rows_v, out_hbm.at[pl.ds(off, chunk)])

    return k(table, idx)


def _lstm_em_body(xf_ref, xb_ref, wif_ref, whf_ref, bf_ref, wib_ref, whb_ref,
                  bb_ref, wof_ref, wob_ref, lens_ref, emf_ref, embr_ref,
                  xwf, xwb, hsf, hsb, hf, cf, hb, cb):
    i = pl.program_id(0)
    B, H = hf.shape
    G = 4 * H
    nstep = xwf.shape[0] // B

    @pl.when(i == 0)
    def _init():
        z = jnp.zeros((B, H), jnp.float32)
        hf[...] = z
        cf[...] = z
        hb[...] = z
        cb[...] = z

    nt = 256
    xfb = xf_ref[...].astype(jnp.bfloat16)
    xbb = xb_ref[...].astype(jnp.bfloat16)
    for j in range(G // nt):
        sl = pl.ds(j * nt, nt)
        xwf[:, sl] = (jnp.dot(xfb, wif_ref[:, sl],
                              preferred_element_type=jnp.float32)
                      + bf_ref[0:1, sl])
        xwb[:, sl] = (jnp.dot(xbb, wib_ref[:, sl],
                              preferred_element_type=jnp.float32)
                      + bb_ref[0:1, sl])

    lens = lens_ref[:, 0:1]

    def step(t, carry):
        tg = (i * nstep + t).astype(jnp.float32)
        m = lens > tg
        rows = pl.ds(t * B, B)

        gf = xwf[rows, :] + jnp.dot(hf[...].astype(jnp.bfloat16),
                                    whf_ref[...],
                                    preferred_element_type=jnp.float32)
        cn = (jax.nn.sigmoid(gf[:, H:2 * H]) * cf[...]
              + jax.nn.sigmoid(gf[:, 0:H]) * jnp.tanh(gf[:, 2 * H:3 * H]))
        hn = jax.nn.sigmoid(gf[:, 3 * H:G]) * jnp.tanh(cn)
        h2 = jnp.where(m, hn, hf[...])
        cf[...] = jnp.where(m, cn, cf[...])
        hf[...] = h2
        hsf[rows, :] = h2.astype(jnp.bfloat16)

        gb = xwb[rows, :] + jnp.dot(hb[...].astype(jnp.bfloat16),
                                    whb_ref[...],
                                    preferred_element_type=jnp.float32)
        cnb = (jax.nn.sigmoid(gb[:, H:2 * H]) * cb[...]
               + jax.nn.sigmoid(gb[:, 0:H]) * jnp.tanh(gb[:, 2 * H:3 * H]))
        hnb = jax.nn.sigmoid(gb[:, 3 * H:G]) * jnp.tanh(cnb)
        h2b = jnp.where(m, hnb, hb[...])
        cb[...] = jnp.where(m, cnb, cb[...])
        hb[...] = h2b
        hsb[rows, :] = h2b.astype(jnp.bfloat16)
        return carry

    lax.fori_loop(0, nstep, step, 0)

    emf_ref[...] = jnp.dot(hsf[...], wof_ref[...],
                           preferred_element_type=jnp.float32)
    embr_ref[...] = jnp.dot(hsb[...], wob_ref[...],
                            preferred_element_type=jnp.float32)


def _lstm_em(x_all, Wif, Whf, bf2, Wib, Whb, bb2, Wof, Wob, lens_b):
    B = lens_b.shape[0]
    E = x_all.shape[1]
    H = Whf.shape[0]
    G = 4 * H
    L = Wof.shape[1]
    T = x_all.shape[0] // (2 * B)
    ct = min(_CT, T)
    nblk = T // ct
    rpb = ct * B

    out_sh = jax.ShapeDtypeStruct((T * B, L), jnp.float32)
    em_f, em_b_r = pl.pallas_call(
        _lstm_em_body,
        grid=(nblk,),
        in_specs=[
            pl.BlockSpec((rpb, E), lambda i: (i, 0)),
            pl.BlockSpec((rpb, E), lambda i, n=nblk: (i + n, 0)),
            pl.BlockSpec((E, G), lambda i: (0, 0)),
            pl.BlockSpec((H, G), lambda i: (0, 0)),
            pl.BlockSpec((B, G), lambda i: (0, 0)),
            pl.BlockSpec((E, G), lambda i: (0, 0)),
            pl.BlockSpec((H, G), lambda i: (0, 0)),
            pl.BlockSpec((B, G), lambda i: (0, 0)),
            pl.BlockSpec((H, L), lambda i: (0, 0)),
            pl.BlockSpec((H, L), lambda i: (0, 0)),
            pl.BlockSpec((B, 128), lambda i: (0, 0)),
        ],
        out_specs=[
            pl.BlockSpec((rpb, L), lambda i: (i, 0)),
            pl.BlockSpec((rpb, L), lambda i: (i, 0)),
        ],
        out_shape=[out_sh, out_sh],
        scratch_shapes=[
            pltpu.VMEM((rpb, G), jnp.float32),
            pltpu.VMEM((rpb, G), jnp.float32),
            pltpu.VMEM((rpb, H), jnp.bfloat16),
            pltpu.VMEM((rpb, H), jnp.bfloat16),
            pltpu.VMEM((B, H), jnp.float32),
            pltpu.VMEM((B, H), jnp.float32),
            pltpu.VMEM((B, H), jnp.float32),
            pltpu.VMEM((B, H), jnp.float32),
        ],
    )(x_all, x_all, Wif, Whf, bf2, Wib, Whb, bb2, Wof, Wob, lens_b)
    return em_f, em_b_r


def _crf_body(emf_ref, emb_ref, expTT_ref, start_ref, end_ref, bout_ref,
              lens_ref, out_ref):
    B = lens_ref.shape[0]
    T = emf_ref.shape[0] // B
    L = start_ref.shape[1]
    M = jnp.exp(expTT_ref[...])
    lens = lens_ref[:, 0:1]

    s = (emf_ref[0:B, 0:L] + emb_ref[0:B, 0:L] + bout_ref[...]
         + start_ref[...])

    def step(t, s):
        rows = pl.ds(t * B, B)
        em_t = emf_ref[rows, 0:L] + emb_ref[rows, 0:L] + bout_ref[...]
        mx = jnp.max(s, axis=1, keepdims=True)
        p = jnp.dot(jnp.exp(s - mx), M, preferred_element_type=jnp.float32)
        new = em_t + mx + jnp.log(p)
        m = lens > t.astype(jnp.float32)
        return jnp.where(m, new, s)

    s = lax.fori_loop(1, T, step, s)
    s = s + end_ref[...]
    mx = jnp.max(s, axis=1, keepdims=True)
    out_ref[...] = mx + jnp.log(jnp.sum(jnp.exp(s - mx), axis=1,
                                        keepdims=True))


def _crf(em_f, em_b, expTT, start_b, end_b, bout_b, lens_b):
    B = lens_b.shape[0]
    L = em_f.shape[1]
    out = pl.pallas_call(
        _crf_body,
        out_shape=jax.ShapeDtypeStruct((B, 1), jnp.float32),
    )(em_f, em_b, expTT, start_b, end_b, bout_b, lens_b)
    return out


def kernel(tokens, lengths, embed, W_ih_f, W_hh_f, b_f, W_ih_b, W_hh_b, b_b,
           W_out, b_out, start_transition, transitions, end_transition):
    B, T = tokens.shape
    H = W_hh_f.shape[0]
    L = transitions.shape[0]

    ar = jnp.arange(T)
    pos = jnp.clip(lengths[:, None] - 1 - ar[None, :], 0, T - 1)
    tokens_r = jnp.take_along_axis(tokens, pos, axis=1)

    idx_f = tokens.T.reshape(-1)
    idx_b = tokens_r.T.reshape(-1)
    idx_all = jnp.concatenate([idx_f, idx_b]).astype(jnp.int32)
    x_all = _sc_gather(embed, idx_all)

    lens_b = jnp.broadcast_to(lengths.astype(jnp.float32)[:, None], (B, 128))
    bf2 = jnp.broadcast_to(b_f[None, :], (B, 4 * H))
    bb2 = jnp.broadcast_to(b_b[None, :], (B, 4 * H))
    lpad = 128 - L
    wof = jnp.pad(W_out[:H], ((0, 0), (0, lpad))).astype(jnp.bfloat16)
    wob = jnp.pad(W_out[H:], ((0, 0), (0, lpad))).astype(jnp.bfloat16)
    em_f, em_b_r = _lstm_em(x_all, W_ih_f.astype(jnp.bfloat16),
                            W_hh_f.astype(jnp.bfloat16), bf2,
                            W_ih_b.astype(jnp.bfloat16),
                            W_hh_b.astype(jnp.bfloat16), bb2,
                            wof, wob, lens_b)

    ridx = (pos.T * B + jnp.arange(B)[None, :]).reshape(-1).astype(jnp.int32)
    em_b = _sc_gather(em_b_r, ridx)

    expTT = transitions.T
    start_b = jnp.broadcast_to(start_transition[None, :], (B, L))
    end_b = jnp.broadcast_to(end_transition[None, :], (B, L))
    bout_b = jnp.broadcast_to(b_out[None, :], (B, L))
    out = _crf(em_f, em_b, expTT, start_b, end_b, bout_b, lens_b)
    return out.reshape(B)


# speedup vs baseline: 1.0680x; 1.0680x over previous
import functools

import jax
import jax.numpy as jnp
from jax import lax
from jax.experimental import pallas as pl
from jax.experimental.pallas import tpu as pltpu
from jax.experimental.pallas import tpu_sc as plsc

_SC_NC = 2
_SC_NS = 16
_SC_NW = _SC_NC * _SC_NS
_CT = 32


def _sc_gather(table, idx):
    n = idx.shape[0]
    d = table.shape[1]
    b_per_w = n // _SC_NW
    chunk = min(128, b_per_w)
    n_chunks = b_per_w // chunk
    mesh = plsc.VectorSubcoreMesh(core_axis_name="c", subcore_axis_name="s")

    @functools.partial(
        pl.kernel, mesh=mesh,
        out_type=jax.ShapeDtypeStruct((n, d), jnp.float32),
        scratch_types=[
            pltpu.VMEM((chunk,), jnp.int32),
            pltpu.VMEM((chunk, d), jnp.float32),
            pltpu.SemaphoreType.DMA,
        ],
    )
    def k(table_hbm, idx_hbm, out_hbm, idx_v, rows_v, sem):
        wid = lax.axis_index("s") * _SC_NC + lax.axis_index("c")
        base = wid * b_per_w
        for j in range(n_chunks):
            off = base + j * chunk
            pltpu.sync_copy(idx_hbm.at[pl.ds(off, chunk)], idx_v)
            pltpu.async_copy(table_hbm.at[idx_v], rows_v, sem).wait()
            pltpu.sync_copy(rows_v, out_hbm.at[pl.ds(off, chunk)])

    return k(table, idx)


def _lstm_em_body(xf_ref, xb_ref, wif_ref, whf_ref, bf_ref, wib_ref, whb_ref,
                  bb_ref, wof_ref, wob_ref, emf_ref, embr_ref,
                  xwf, xwb, hsf, hsb, hc):
    i = pl.program_id(0)
    B = hc.shape[1]
    H = hc.shape[2]
    G = 4 * H
    nstep = xwf.shape[0] // B

    @pl.when(i == 0)
    def _init():
        hc[...] = jnp.zeros((4, B, H), jnp.float32)

    nt = 256
    xfb = xf_ref[...].astype(jnp.bfloat16)
    xbb = xb_ref[...].astype(jnp.bfloat16)
    for j in range(G // nt):
        sl = pl.ds(j * nt, nt)
        xwf[:, sl] = (jnp.dot(xfb, wif_ref[:, sl],
                              preferred_element_type=jnp.float32)
                      + bf_ref[0:1, sl])
        xwb[:, sl] = (jnp.dot(xbb, wib_ref[:, sl],
                              preferred_element_type=jnp.float32)
                      + bb_ref[0:1, sl])

    def step(t, carry):
        hf, cf, hb, cb = carry
        rows = pl.ds(t * B, B)

        gf = xwf[rows, :] + jnp.dot(hf.astype(jnp.bfloat16), whf_ref[...],
                                    preferred_element_type=jnp.float32)
        gb = xwb[rows, :] + jnp.dot(hb.astype(jnp.bfloat16), whb_ref[...],
                                    preferred_element_type=jnp.float32)
        cnf = (jax.nn.sigmoid(gf[:, H:2 * H]) * cf
               + jax.nn.sigmoid(gf[:, 0:H]) * jnp.tanh(gf[:, 2 * H:3 * H]))
        hnf = jax.nn.sigmoid(gf[:, 3 * H:G]) * jnp.tanh(cnf)
        hsf[rows, :] = hnf
        cnb = (jax.nn.sigmoid(gb[:, H:2 * H]) * cb
               + jax.nn.sigmoid(gb[:, 0:H]) * jnp.tanh(gb[:, 2 * H:3 * H]))
        hnb = jax.nn.sigmoid(gb[:, 3 * H:G]) * jnp.tanh(cnb)
        hsb[rows, :] = hnb
        return (hnf, cnf, hnb, cnb)

    carry = (hc[0], hc[1], hc[2], hc[3])
    carry = lax.fori_loop(0, nstep, step, carry, unroll=2)
    hc[0] = carry[0]
    hc[1] = carry[1]
    hc[2] = carry[2]
    hc[3] = carry[3]

    emf_ref[...] = jnp.dot(hsf[...].astype(jnp.bfloat16), wof_ref[...],
                           preferred_element_type=jnp.float32)
    embr_ref[...] = jnp.dot(hsb[...].astype(jnp.bfloat16), wob_ref[...],
                            preferred_element_type=jnp.float32)


def _lstm_em(x_all, Wif, Whf, bf2, Wib, Whb, bb2, Wof, Wob, B):
    E = x_all.shape[1]
    H = Whf.shape[0]
    G = 4 * H
    L = Wof.shape[1]
    T = x_all.shape[0] // (2 * B)
    ct = min(_CT, T)
    nblk = T // ct
    rpb = ct * B

    out_sh = jax.ShapeDtypeStruct((T * B, L), jnp.float32)
    em_f, em_b_r = pl.pallas_call(
        _lstm_em_body,
        grid=(nblk,),
        in_specs=[
            pl.BlockSpec((rpb, E), lambda i: (i, 0)),
            pl.BlockSpec((rpb, E), lambda i, n=nblk: (i + n, 0)),
            pl.BlockSpec((E, G), lambda i: (0, 0)),
            pl.BlockSpec((H, G), lambda i: (0, 0)),
            pl.BlockSpec((B, G), lambda i: (0, 0)),
            pl.BlockSpec((E, G), lambda i: (0, 0)),
            pl.BlockSpec((H, G), lambda i: (0, 0)),
            pl.BlockSpec((B, G), lambda i: (0, 0)),
            pl.BlockSpec((H, L), lambda i: (0, 0)),
            pl.BlockSpec((H, L), lambda i: (0, 0)),
        ],
        out_specs=[
            pl.BlockSpec((rpb, L), lambda i: (i, 0)),
            pl.BlockSpec((rpb, L), lambda i: (i, 0)),
        ],
        out_shape=[out_sh, out_sh],
        scratch_shapes=[
            pltpu.VMEM((rpb, G), jnp.float32),
            pltpu.VMEM((rpb, G), jnp.float32),
            pltpu.VMEM((rpb, H), jnp.float32),
            pltpu.VMEM((rpb, H), jnp.float32),
            pltpu.VMEM((4, B, H), jnp.float32),
        ],
    )(x_all, x_all, Wif, Whf, bf2, Wib, Whb, bb2, Wof, Wob)
    return em_f, em_b_r


def _crf_body(emf_ref, emb_ref, expTT_ref, start_ref, end_ref, bout_ref,
              lens_ref, out_ref):
    B = lens_ref.shape[0]
    T = emf_ref.shape[0] // B
    L = start_ref.shape[1]
    M = jnp.exp(expTT_ref[...])
    lens = lens_ref[:, 0:1]

    s = (emf_ref[0:B, 0:L] + emb_ref[0:B, 0:L] + bout_ref[...]
         + start_ref[...])

    def step(t, s):
        rows = pl.ds(t * B, B)
        em_t = emf_ref[rows, 0:L] + emb_ref[rows, 0:L] + bout_ref[...]
        mx = jnp.max(s, axis=1, keepdims=True)
        p = jnp.dot(jnp.exp(s - mx), M, preferred_element_type=jnp.float32)
        new = em_t + mx + jnp.log(p)
        m = lens > t.astype(jnp.float32)
        return jnp.where(m, new, s)

    s = lax.fori_loop(1, T, step, s)
    s = s + end_ref[...]
    mx = jnp.max(s, axis=1, keepdims=True)
    out_ref[...] = mx + jnp.log(jnp.sum(jnp.exp(s - mx), axis=1,
                                        keepdims=True))


def _crf(em_f, em_b, expTT, start_b, end_b, bout_b, lens_b):
    B = lens_b.shape[0]
    L = em_f.shape[1]
    out = pl.pallas_call(
        _crf_body,
        out_shape=jax.ShapeDtypeStruct((B, 1), jnp.float32),
    )(em_f, em_b, expTT, start_b, end_b, bout_b, lens_b)
    return out


def kernel(tokens, lengths, embed, W_ih_f, W_hh_f, b_f, W_ih_b, W_hh_b, b_b,
           W_out, b_out, start_transition, transitions, end_transition):
    B, T = tokens.shape
    H = W_hh_f.shape[0]
    L = transitions.shape[0]

    ar = jnp.arange(T)
    pos = jnp.clip(lengths[:, None] - 1 - ar[None, :], 0, T - 1)
    tokens_r = jnp.take_along_axis(tokens, pos, axis=1)

    idx_f = tokens.T.reshape(-1)
    idx_b = tokens_r.T.reshape(-1)
    idx_all = jnp.concatenate([idx_f, idx_b]).astype(jnp.int32)
    x_all = _sc_gather(embed, idx_all)

    lens_b = jnp.broadcast_to(lengths.astype(jnp.float32)[:, None], (B, 128))
    bf2 = jnp.broadcast_to(b_f[None, :], (B, 4 * H))
    bb2 = jnp.broadcast_to(b_b[None, :], (B, 4 * H))
    lpad = 128 - L
    wof = jnp.pad(W_out[:H], ((0, 0), (0, lpad))).astype(jnp.bfloat16)
    wob = jnp.pad(W_out[H:], ((0, 0), (0, lpad))).astype(jnp.bfloat16)
    em_f, em_b_r = _lstm_em(x_all, W_ih_f.astype(jnp.bfloat16),
                            W_hh_f.astype(jnp.bfloat16), bf2,
                            W_ih_b.astype(jnp.bfloat16),
                            W_hh_b.astype(jnp.bfloat16), bb2,
                            wof, wob, B)

    ridx = (pos.T * B + jnp.arange(B)[None, :]).reshape(-1).astype(jnp.int32)
    em_b = _sc_gather(em_b_r, ridx)

    expTT = transitions.T
    start_b = jnp.broadcast_to(start_transition[None, :], (B, L))
    end_b = jnp.broadcast_to(end_transition[None, :], (B, L))
    bout_b = jnp.broadcast_to(b_out[None, :], (B, L))
    out = _crf(em_f, em_b, expTT, start_b, end_b, bout_b, lens_b)
    return out.reshape(B)


# speedup vs baseline: 1.1538x; 1.0804x over previous
import functools

import jax
import jax.numpy as jnp
from jax import lax
from jax.experimental import pallas as pl
from jax.experimental.pallas import tpu as pltpu
from jax.experimental.pallas import tpu_sc as plsc

_SC_NC = 2
_SC_NS = 16
_SC_NW = _SC_NC * _SC_NS
_CT = 32


def _sc_gather(table, idx):
    n = idx.shape[0]
    d = table.shape[1]
    b_per_w = n // _SC_NW
    chunk = min(128, b_per_w)
    n_chunks = b_per_w // chunk
    mesh = plsc.VectorSubcoreMesh(core_axis_name="c", subcore_axis_name="s")

    @functools.partial(
        pl.kernel, mesh=mesh,
        out_type=jax.ShapeDtypeStruct((n, d), jnp.float32),
        scratch_types=[
            pltpu.VMEM((chunk,), jnp.int32),
            pltpu.VMEM((chunk, d), jnp.float32),
            pltpu.SemaphoreType.DMA,
        ],
    )
    def k(table_hbm, idx_hbm, out_hbm, idx_v, rows_v, sem):
        wid = lax.axis_index("s") * _SC_NC + lax.axis_index("c")
        base = wid * b_per_w
        for j in range(n_chunks):
            off = base + j * chunk
            pltpu.sync_copy(idx_hbm.at[pl.ds(off, chunk)], idx_v)
            pltpu.async_copy(table_hbm.at[idx_v], rows_v, sem).wait()
            pltpu.sync_copy(rows_v, out_hbm.at[pl.ds(off, chunk)])

    return k(table, idx)


def _lstm_em_body(xf_ref, xb_ref, wif_ref, whf_ref, bf_ref, wib_ref, whb_ref,
                  bb_ref, wof_ref, wob_ref, emf_ref, embr_ref,
                  xwf, xwb, hsf, hsb, hc):
    i = pl.program_id(0)
    B = hc.shape[1]
    H = hc.shape[2]
    G = 4 * H
    nstep = xwf.shape[0] // B

    @pl.when(i == 0)
    def _init():
        hc[...] = jnp.zeros((4, B, H), jnp.float32)

    nt = 256
    xfb = xf_ref[...].astype(jnp.bfloat16)
    xbb = xb_ref[...].astype(jnp.bfloat16)
    for j in range(G // nt):
        sl = pl.ds(j * nt, nt)
        xwf[:, sl] = (jnp.dot(xfb, wif_ref[:, sl],
                              preferred_element_type=jnp.float32)
                      + bf_ref[0:1, sl])
        xwb[:, sl] = (jnp.dot(xbb, wib_ref[:, sl],
                              preferred_element_type=jnp.float32)
                      + bb_ref[0:1, sl])

    def step(t, carry):
        hf, cf, hb, cb = carry
        rows = pl.ds(t * B, B)

        gf = xwf[rows, :] + jnp.dot(hf.astype(jnp.bfloat16), whf_ref[...],
                                    preferred_element_type=jnp.float32)
        gb = xwb[rows, :] + jnp.dot(hb.astype(jnp.bfloat16), whb_ref[...],
                                    preferred_element_type=jnp.float32)
        cnf = (jax.nn.sigmoid(gf[:, H:2 * H]) * cf
               + jax.nn.sigmoid(gf[:, 0:H]) * jnp.tanh(gf[:, 2 * H:3 * H]))
        hnf = jax.nn.sigmoid(gf[:, 3 * H:G]) * jnp.tanh(cnf)
        hsf[rows, :] = hnf
        cnb = (jax.nn.sigmoid(gb[:, H:2 * H]) * cb
               + jax.nn.sigmoid(gb[:, 0:H]) * jnp.tanh(gb[:, 2 * H:3 * H]))
        hnb = jax.nn.sigmoid(gb[:, 3 * H:G]) * jnp.tanh(cnb)
        hsb[rows, :] = hnb
        return (hnf, cnf, hnb, cnb)

    carry = (hc[0], hc[1], hc[2], hc[3])
    carry = lax.fori_loop(0, nstep, step, carry, unroll=2)
    hc[0] = carry[0]
    hc[1] = carry[1]
    hc[2] = carry[2]
    hc[3] = carry[3]

    emf_ref[...] = jnp.dot(hsf[...].astype(jnp.bfloat16), wof_ref[...],
                           preferred_element_type=jnp.float32)
    embr_ref[...] = jnp.dot(hsb[...].astype(jnp.bfloat16), wob_ref[...],
                            preferred_element_type=jnp.float32)


def _lstm_em(x_all, Wif, Whf, bf2, Wib, Whb, bb2, Wof, Wob, B):
    E = x_all.shape[1]
    H = Whf.shape[0]
    G = 4 * H
    L = Wof.shape[1]
    T = x_all.shape[0] // (2 * B)
    ct = min(_CT, T)
    nblk = T // ct
    rpb = ct * B

    out_sh = jax.ShapeDtypeStruct((T * B, L), jnp.float32)
    em_f, em_b_r = pl.pallas_call(
        _lstm_em_body,
        grid=(nblk,),
        in_specs=[
            pl.BlockSpec((rpb, E), lambda i: (i, 0)),
            pl.BlockSpec((rpb, E), lambda i, n=nblk: (i + n, 0)),
            pl.BlockSpec((E, G), lambda i: (0, 0)),
            pl.BlockSpec((H, G), lambda i: (0, 0)),
            pl.BlockSpec((B, G), lambda i: (0, 0)),
            pl.BlockSpec((E, G), lambda i: (0, 0)),
            pl.BlockSpec((H, G), lambda i: (0, 0)),
            pl.BlockSpec((B, G), lambda i: (0, 0)),
            pl.BlockSpec((H, L), lambda i: (0, 0)),
            pl.BlockSpec((H, L), lambda i: (0, 0)),
        ],
        out_specs=[
            pl.BlockSpec((rpb, L), lambda i: (i, 0)),
            pl.BlockSpec((rpb, L), lambda i: (i, 0)),
        ],
        out_shape=[out_sh, out_sh],
        scratch_shapes=[
            pltpu.VMEM((rpb, G), jnp.float32),
            pltpu.VMEM((rpb, G), jnp.float32),
            pltpu.VMEM((rpb, H), jnp.float32),
            pltpu.VMEM((rpb, H), jnp.float32),
            pltpu.VMEM((4, B, H), jnp.float32),
        ],
    )(x_all, x_all, Wif, Whf, bf2, Wib, Whb, bb2, Wof, Wob)
    return em_f, em_b_r


def _crf_body(emf_ref, emb_ref, expTT_ref, start_ref, end_ref, bout_ref,
              lens_ref, out_ref, E, D):
    B = lens_ref.shape[0]
    TB = emf_ref.shape[0]
    T = TB // B
    L = start_ref.shape[1]
    M = jnp.exp(expTT_ref[...])
    lens = lens_ref[:, 0:1]
    bout_row = bout_ref[0:1, :]

    chunk = 256
    for j in range(TB // chunk):
        rows = pl.ds(j * chunk, chunk)
        em = (emf_ref[rows, 0:L] + emb_ref[rows, 0:L]
              + jnp.broadcast_to(bout_row, (chunk, L)))
        d = jnp.max(em, axis=1, keepdims=True) + 2.08
        E[rows, :] = jnp.exp(em - d)
        D[rows, :] = d

    s0 = (emf_ref[0:B, 0:L] + emb_ref[0:B, 0:L] + bout_ref[...]
          + start_ref[...])
    mx0 = jnp.max(s0, axis=1, keepdims=True)
    w0 = jnp.exp(s0 - mx0)

    def step(t, carry):
        w, acc = carry
        rows = pl.ds(t * B, B)
        inv_r = 1.0 / jnp.max(w, axis=1, keepdims=True)
        p = jnp.dot(w, M, preferred_element_type=jnp.float32)
        wn = E[rows, :] * p * inv_r
        accn = acc + D[rows, :] - jnp.log(inv_r)
        m = lens > t.astype(jnp.float32)
        return (jnp.where(m, wn, w), jnp.where(m, accn, acc))

    w, acc = lax.fori_loop(1, T, step, (w0, mx0), unroll=2)
    q = w * jnp.exp(end_ref[...])
    out_ref[...] = jnp.log(jnp.sum(q, axis=1, keepdims=True)) + acc


def _crf(em_f, em_b, expTT, start_b, end_b, bout_b, lens_b):
    B = lens_b.shape[0]
    L = start_b.shape[1]
    TB = em_f.shape[0]
    out = pl.pallas_call(
        _crf_body,
        out_shape=jax.ShapeDtypeStruct((B, 1), jnp.float32),
        scratch_shapes=[
            pltpu.VMEM((TB, L), jnp.float32),
            pltpu.VMEM((TB, 1), jnp.float32),
        ],
    )(em_f, em_b, expTT, start_b, end_b, bout_b, lens_b)
    return out


def kernel(tokens, lengths, embed, W_ih_f, W_hh_f, b_f, W_ih_b, W_hh_b, b_b,
           W_out, b_out, start_transition, transitions, end_transition):
    B, T = tokens.shape
    H = W_hh_f.shape[0]
    L = transitions.shape[0]

    ar = jnp.arange(T)
    pos = jnp.clip(lengths[:, None] - 1 - ar[None, :], 0, T - 1)
    tokens_r = jnp.take_along_axis(tokens, pos, axis=1)

    idx_f = tokens.T.reshape(-1)
    idx_b = tokens_r.T.reshape(-1)
    idx_all = jnp.concatenate([idx_f, idx_b]).astype(jnp.int32)
    x_all = _sc_gather(embed, idx_all)

    lens_b = jnp.broadcast_to(lengths.astype(jnp.float32)[:, None], (B, 128))
    bf2 = jnp.broadcast_to(b_f[None, :], (B, 4 * H))
    bb2 = jnp.broadcast_to(b_b[None, :], (B, 4 * H))
    lpad = 128 - L
    wof = jnp.pad(W_out[:H], ((0, 0), (0, lpad))).astype(jnp.bfloat16)
    wob = jnp.pad(W_out[H:], ((0, 0), (0, lpad))).astype(jnp.bfloat16)
    em_f, em_b_r = _lstm_em(x_all, W_ih_f.astype(jnp.bfloat16),
                            W_hh_f.astype(jnp.bfloat16), bf2,
                            W_ih_b.astype(jnp.bfloat16),
                            W_hh_b.astype(jnp.bfloat16), bb2,
                            wof, wob, B)

    ridx = (pos.T * B + jnp.arange(B)[None, :]).reshape(-1).astype(jnp.int32)
    em_b = _sc_gather(em_b_r, ridx)

    expTT = transitions.T
    start_b = jnp.broadcast_to(start_transition[None, :], (B, L))
    end_b = jnp.broadcast_to(end_transition[None, :], (B, L))
    bout_b = jnp.broadcast_to(b_out[None, :], (B, L))
    out = _crf(em_f, em_b, expTT, start_b, end_b, bout_b, lens_b)
    return out.reshape(B)
